# Initial kernel scaffold; baseline (speedup 1.0000x reference)
import jax
import jax.numpy as jnp
from jax.experimental import pallas as pl


def kernel(idx, token_embedding_table):
    raise NotImplementedError("write your pallas kernel here")


# speedup vs baseline: 2.9445x; 2.9445x over previous
import functools

import jax
import jax.numpy as jnp
from jax import lax
from jax.experimental import pallas as pl
from jax.experimental.pallas import tpu as pltpu
from jax.experimental.pallas import tpu_sc as plsc

_VOCAB = 64
_T = 50
_D = 32
_B = 16384
_ROWS = _B * _T
_GCOLS = 128
_GROWS = _ROWS // _GCOLS

_NC = 2
_NS = 16
_NW = _NC * _NS
_STEPS_PER_TILE = _ROWS // (_NW * _GCOLS)
_NBUF = 8
_DEPTH = 4


def _prep_body(tab_ref, idx_ref, c_ref, g_ref):
    tab = tab_ref[...]
    pos = tab_ref[0:_T, :]
    c_ref[...] = pos[:, None, :] + tab[None, :, :]
    r = lax.broadcasted_iota(jnp.int32, (_GROWS, _GCOLS), 0)
    c = lax.broadcasted_iota(jnp.int32, (_GROWS, _GCOLS), 1)
    k = r * _GCOLS + c
    g_ref[...] = idx_ref[...] + (k % _T) * _VOCAB


_prep = pl.pallas_call(
    _prep_body,
    out_shape=[
        jax.ShapeDtypeStruct((_T, _VOCAB, _D), jnp.float32),
        jax.ShapeDtypeStruct((_GROWS, _GCOLS), jnp.int32),
    ],
)


def _gather_body(c_hbm, g_hbm, out_hbm, gbuf, bufs, gsem, ssem):
    cid = lax.axis_index("c")
    sid = lax.axis_index("s")
    wid = sid * _NC + cid
    gbase = wid * _STEPS_PER_TILE
    rowbase = gbase * _GCOLS

    pltpu.sync_copy(g_hbm.at[pl.ds(gbase, _STEPS_PER_TILE)], gbuf)

    for j in range(_DEPTH):
        pltpu.async_copy(c_hbm.at[gbuf.at[j]], bufs.at[j], gsem.at[j])

    def outer(io, carry):
        for bo in range(_NBUF):
            i = io * _NBUF + bo
            pltpu.make_async_copy(
                c_hbm.at[gbuf.at[i]], bufs.at[bo], gsem.at[bo]
            ).wait()
            pltpu.async_copy(
                bufs.at[bo],
                out_hbm.at[pl.ds(rowbase + i * _GCOLS, _GCOLS)],
                ssem.at[bo],
            )
            nb = (bo + _DEPTH) % _NBUF

            @pl.when(i >= _DEPTH)
            def _wait_store():
                pltpu.make_async_copy(
                    bufs.at[nb],
                    out_hbm.at[pl.ds(rowbase, _GCOLS)],
                    ssem.at[nb],
                ).wait()

            @pl.when(i < _STEPS_PER_TILE - _DEPTH)
            def _fire_gather():
                pltpu.async_copy(
                    c_hbm.at[gbuf.at[i + _DEPTH]], bufs.at[nb], gsem.at[nb]
                )

        return carry

    lax.fori_loop(0, _STEPS_PER_TILE // _NBUF, outer, 0)

    for i in range(_STEPS_PER_TILE - _DEPTH, _STEPS_PER_TILE):
        bo = i % _NBUF
        pltpu.make_async_copy(
            bufs.at[bo],
            out_hbm.at[pl.ds(rowbase, _GCOLS)],
            ssem.at[bo],
        ).wait()


_gather = pl.kernel(
    _gather_body,
    out_type=jax.ShapeDtypeStruct((_ROWS, _D), jnp.float32),
    mesh=plsc.VectorSubcoreMesh(core_axis_name="c", subcore_axis_name="s"),
    scratch_types=[
        pltpu.VMEM((_STEPS_PER_TILE, _GCOLS), jnp.int32),
        pltpu.VMEM((_NBUF, _GCOLS, _D), jnp.float32),
        pltpu.SemaphoreType.DMA((_NBUF,)),
        pltpu.SemaphoreType.DMA((_NBUF,)),
    ],
    compiler_params=pltpu.CompilerParams(use_tc_tiling_on_sc=False),
)


def kernel(idx, token_embedding_table):
    B, T = idx.shape
    idx2d = idx.reshape(_GROWS, _GCOLS).astype(jnp.int32)
    c3, g = _prep(token_embedding_table, idx2d)
    c = c3.reshape(_T * _VOCAB, _D)
    out = _gather(c, g)
    return out.reshape(B, T, _D)


# speedup vs baseline: 4.2901x; 1.4570x over previous
import jax
import jax.numpy as jnp
from jax import lax
from jax.experimental import pallas as pl
from jax.experimental.pallas import tpu as pltpu
from jax.experimental.pallas import tpu_sc as plsc

_VOCAB = 64
_T = 50
_TM = 48
_D = 32
_B = 16384
_ROWS = _B * _T
_GCOLS = 128

_NC = 2
_NS = 16
_NW = _NC * _NS
_BPW = _B // _NW
_IPW = _BPW * _T
_MB = 4
_SEC = _MB * (_TM + 16)
_NSTEPS = _BPW // _MB
_NBUF = 2
_L = 16


def _prep_body(tab_ref, tab16_ref, c_ref):
    tab16 = tab16_ref[...]
    pos = tab_ref[0:_T, :]
    pos128 = jnp.concatenate([pos, pos, pos, pos], axis=1)
    c = tab16[None, :, :] + pos128[:, None, :]
    c_ref[...] = c.reshape(_T * 16, _GCOLS)


_prep = pl.pallas_call(
    _prep_body,
    out_shape=jax.ShapeDtypeStruct((_T * 16, _GCOLS), jnp.float32),
)


def _gather_body(c_hbm, idx_hbm, out_hbm, gbuf, bufs, tbuf, gsem, ssem):
    cid = lax.axis_index("c")
    sid = lax.axis_index("s")
    wid = sid * _NC + cid
    bbase = wid * _BPW

    pltpu.sync_copy(idx_hbm.at[pl.ds(wid * _NSTEPS * _SEC, _NSTEPS * _SEC)], gbuf)

    lane = lax.broadcasted_iota(jnp.int32, (_L,), 0)
    toff = [(c * _L + lane) * _VOCAB for c in range(3)]
    tail_off = jnp.minimum(_TM + lane, _T - 1) * _VOCAB
    sec_pat = []
    for k in range(_MB):
        sec_pat.extend(toff)
    for k in range(_MB):
        sec_pat.append(tail_off)

    def addpos(m, carry):
        for q in range(_SEC // _L):
            off = m * _SEC + q * _L
            gbuf[pl.ds(off, _L)] = gbuf[pl.ds(off, _L)] + sec_pat[q]
        return carry

    lax.fori_loop(0, _NSTEPS, addpos, 0)

    def outer(io, carry):
        for bo in range(_NBUF):
            m = io * _NBUF + bo

            @pl.when(m >= _NBUF)
            def _wait_store():
                pltpu.make_async_copy(
                    bufs.at[bo],
                    out_hbm.at[pl.ds(bbase, _MB)],
                    ssem.at[bo],
                ).wait()

            for k in range(_MB):
                pltpu.async_copy(
                    c_hbm.at[gbuf.at[pl.ds(m * _SEC + k * _TM, _TM)]],
                    bufs.at[bo, k, pl.ds(0, _TM)],
                    gsem.at[bo],
                )
            for k in range(_MB):
                pltpu.async_copy(
                    c_hbm.at[
                        gbuf.at[pl.ds(m * _SEC + _MB * _TM + k * _L, _L)]
                    ],
                    tbuf.at[bo, k],
                    gsem.at[bo],
                )
            pltpu.make_async_copy(
                out_hbm.at[pl.ds(bbase, _MB), pl.ds(0, _TM)],
                bufs.at[bo, pl.ds(0, _MB), pl.ds(0, _TM)],
                gsem.at[bo],
            ).wait()
            pltpu.make_async_copy(
                out_hbm.at[pl.ds(bbase, _MB), pl.ds(0, _L)],
                tbuf.at[bo],
                gsem.at[bo],
            ).wait()
            for k in range(_MB):
                for r in range(2):
                    for h in range(2):
                        bufs[bo, k, _TM + r, pl.ds(h * _L, _L)] = tbuf[
                            bo, k, r, pl.ds(h * _L, _L)
                        ]
            pltpu.async_copy(
                bufs.at[bo],
                out_hbm.at[pl.ds(bbase + m * _MB, _MB)],
                ssem.at[bo],
            )
        return carry

    lax.fori_loop(0, _NSTEPS // _NBUF, outer, 0)

    for bo in range(_NBUF):
        pltpu.make_async_copy(
            bufs.at[bo],
            out_hbm.at[pl.ds(bbase, _MB)],
            ssem.at[bo],
        ).wait()


_gather = pl.kernel(
    _gather_body,
    out_type=jax.ShapeDtypeStruct((_B, _T, _D), jnp.float32),
    mesh=plsc.VectorSubcoreMesh(core_axis_name="c", subcore_axis_name="s"),
    scratch_types=[
        pltpu.VMEM((_NSTEPS * _SEC,), jnp.int32),
        pltpu.VMEM((_NBUF, _MB, _T, _D), jnp.float32),
        pltpu.VMEM((_NBUF, _MB, _L, _D), jnp.float32),
        pltpu.SemaphoreType.DMA((_NBUF,)),
        pltpu.SemaphoreType.DMA((_NBUF,)),
    ],
    compiler_params=pltpu.CompilerParams(use_tc_tiling_on_sc=False),
)


def kernel(idx, token_embedding_table):
    B, T = idx.shape
    idx = idx.astype(jnp.int32)
    main = idx[:, :_TM].reshape(_B // _MB, _MB * _TM)
    tails = jnp.concatenate(
        [idx[:, _TM:_T], jnp.broadcast_to(idx[:, _T - 1 :], (_B, _L - 2))],
        axis=1,
    ).reshape(_B // _MB, _MB * _L)
    gstream = jnp.concatenate([main, tails], axis=1).reshape(-1)
    tab16 = token_embedding_table.reshape(16, _GCOLS)
    c_lin = _prep(token_embedding_table, tab16)
    c = c_lin.reshape(_T * _VOCAB, _D)
    return _gather(c, gstream)


# speedup vs baseline: 5.8442x; 1.3623x over previous
import jax
import jax.numpy as jnp
from jax import lax
from jax.experimental import pallas as pl
from jax.experimental.pallas import tpu as pltpu
from jax.experimental.pallas import tpu_sc as plsc

_VOCAB = 64
_L = 16
_T = 50
_TM = 48
_D = 32
_B = 16384
_ROWS = _B * _T
_GCOLS = 128

_NC = 2
_NS = 16
_NW = _NC * _NS
_BPW = _B // _NW
_IPW = _BPW * _T
_MB = 8
_SEC = _MB * _TM + _L
_NSTEPS = _BPW // _MB
_NBUF = 3


def _prep_body(tab_ref, tab16_ref, c_ref):
    tab16 = tab16_ref[...]
    pos = tab_ref[0:_T, :]
    pos128 = jnp.concatenate([pos, pos, pos, pos], axis=1)
    c = tab16[None, :, :] + pos128[:, None, :]
    c_ref[...] = c.reshape(_T * 16, _GCOLS)


_prep = pl.pallas_call(
    _prep_body,
    out_shape=jax.ShapeDtypeStruct((_T * 16, _GCOLS), jnp.float32),
)


def _gather_body(c_hbm, idx_hbm, out_hbm, gbuf, bufs, tbuf, gsem, ssem):
    cid = lax.axis_index("c")
    sid = lax.axis_index("s")
    wid = sid * _NC + cid
    bbase = wid * _BPW

    pltpu.sync_copy(idx_hbm.at[pl.ds(wid * _NSTEPS * _SEC, _NSTEPS * _SEC)], gbuf)

    lane = lax.broadcasted_iota(jnp.int32, (_L,), 0)
    toff = [(c * _L + lane) * _VOCAB for c in range(3)]
    tail_off = (_TM + (lane & 1)) * _VOCAB
    sec_pat = []
    for k in range(_MB):
        sec_pat.extend(toff)
    sec_pat.append(tail_off)

    def addpos(m, carry):
        for q in range(_SEC // _L):
            off = m * _SEC + q * _L
            gbuf[pl.ds(off, _L)] = gbuf[pl.ds(off, _L)] + sec_pat[q]
        return carry

    lax.fori_loop(0, _NSTEPS, addpos, 0)

    def outer(m, carry):
        bo = lax.rem(m, _NBUF)

        @pl.when(m >= _NBUF)
        def _wait_store():
            pltpu.make_async_copy(
                bufs.at[bo],
                out_hbm.at[pl.ds(bbase, _MB)],
                ssem.at[bo],
            ).wait()

        for k in range(_MB):
            pltpu.async_copy(
                c_hbm.at[gbuf.at[pl.ds(m * _SEC + k * _TM, _TM)]],
                bufs.at[bo, k, pl.ds(0, _TM)],
                gsem.at[bo],
            )
        pltpu.async_copy(
            c_hbm.at[gbuf.at[pl.ds(m * _SEC + _MB * _TM, _L)]],
            tbuf.at[bo],
            gsem.at[bo],
        )
        pltpu.make_async_copy(
            out_hbm.at[pl.ds(bbase, _MB), pl.ds(0, _TM)],
            bufs.at[bo, pl.ds(0, _MB), pl.ds(0, _TM)],
            gsem.at[bo],
        ).wait()
        pltpu.make_async_copy(
            out_hbm.at[pl.ds(bbase, 1), pl.ds(0, _L)],
            tbuf.at[bo],
            gsem.at[bo],
        ).wait()
        for k in range(_MB):
            for r in range(2):
                for h in range(2):
                    bufs[bo, k, _TM + r, pl.ds(h * _L, _L)] = tbuf[
                        bo, 2 * k + r, pl.ds(h * _L, _L)
                    ]
        pltpu.async_copy(
            bufs.at[bo],
            out_hbm.at[pl.ds(bbase + m * _MB, _MB)],
            ssem.at[bo],
        )
        return carry

    lax.fori_loop(0, _NSTEPS, outer, 0)

    for bo in range(_NBUF):
        pltpu.make_async_copy(
            bufs.at[bo],
            out_hbm.at[pl.ds(bbase, _MB)],
            ssem.at[bo],
        ).wait()


_gather = pl.kernel(
    _gather_body,
    out_type=jax.ShapeDtypeStruct((_B, _T, _D), jnp.float32),
    mesh=plsc.VectorSubcoreMesh(core_axis_name="c", subcore_axis_name="s"),
    scratch_types=[
        pltpu.VMEM((_NSTEPS * _SEC,), jnp.int32),
        pltpu.VMEM((_NBUF, _MB, _T, _D), jnp.float32),
        pltpu.VMEM((_NBUF, _L, _D), jnp.float32),
        pltpu.SemaphoreType.DMA((_NBUF,)),
        pltpu.SemaphoreType.DMA((_NBUF,)),
    ],
    compiler_params=pltpu.CompilerParams(use_tc_tiling_on_sc=False),
)


def kernel(idx, token_embedding_table):
    B, T = idx.shape
    idx = idx.astype(jnp.int32)
    main = idx[:, :_TM].reshape(_B // _MB, _MB * _TM)
    tails = idx[:, _TM:_T].reshape(_B // _MB, _MB * 2)
    gstream = jnp.concatenate([main, tails], axis=1).reshape(-1)
    tab16 = token_embedding_table.reshape(16, _GCOLS)
    c_lin = _prep(token_embedding_table, tab16)
    c = c_lin.reshape(_T * _VOCAB, _D)
    return _gather(c, gstream)


# speedup vs baseline: 5.9060x; 1.0106x over previous
import jax
import jax.numpy as jnp
from jax import lax
from jax.experimental import pallas as pl
from jax.experimental.pallas import tpu as pltpu
from jax.experimental.pallas import tpu_sc as plsc

_VOCAB = 64
_L = 16
_T = 50
_TM = 48
_D = 32
_B = 16384
_ROWS = _B * _T
_GCOLS = 128

_NC = 2
_NS = 16
_NW = _NC * _NS
_BPW = _B // _NW
_IPW = _BPW * _T
_MB = 8
_SEC = _MB * _TM + _L
_NSTEPS = _BPW // _MB
_NBUF = 3


def _prep_body(tab_ref, tab16_ref, c_ref):
    tab16 = tab16_ref[...]
    pos = tab_ref[0:_T, :]
    pos128 = jnp.concatenate([pos, pos, pos, pos], axis=1)
    c = tab16[None, :, :] + pos128[:, None, :]
    c_ref[...] = c.reshape(_T * 16, _GCOLS)


_prep = pl.pallas_call(
    _prep_body,
    out_shape=jax.ShapeDtypeStruct((_T * 16, _GCOLS), jnp.float32),
)


def _gather_body(c_hbm, idx_hbm, out_hbm, gbuf, bufs, tbuf, gsem, ssem):
    cid = lax.axis_index("c")
    sid = lax.axis_index("s")
    wid = sid * _NC + cid
    bbase = wid * _BPW

    pltpu.sync_copy(idx_hbm.at[pl.ds(wid * _NSTEPS * _SEC, _NSTEPS * _SEC)], gbuf)

    lane = lax.broadcasted_iota(jnp.int32, (_L,), 0)
    toff = [(c * _L + lane) * _VOCAB for c in range(3)]
    tail_off = (_TM + (lane & 1)) * _VOCAB
    sec_pat = []
    for k in range(_MB):
        sec_pat.extend(toff)
    sec_pat.append(tail_off)

    def addpos(m, carry):
        for q in range(_SEC // _L):
            off = m * _SEC + q * _L
            gbuf[pl.ds(off, _L)] = gbuf[pl.ds(off, _L)] + sec_pat[q]
        return carry

    lax.fori_loop(0, _NSTEPS, addpos, 0)

    def fire_gathers(m, bo):
        for k in range(_MB):
            pltpu.async_copy(
                c_hbm.at[gbuf.at[pl.ds(m * _SEC + k * _TM, _TM)]],
                bufs.at[bo, k, pl.ds(0, _TM)],
                gsem.at[bo],
            )
        pltpu.async_copy(
            c_hbm.at[gbuf.at[pl.ds(m * _SEC + _MB * _TM, _L)]],
            tbuf.at[bo],
            gsem.at[bo],
        )

    fire_gathers(0, 0)

    def outer(m, carry):
        bo = lax.rem(m, _NBUF)
        nbo = lax.rem(m + 1, _NBUF)

        @pl.when(jnp.logical_and(m + 1 < _NSTEPS, m + 1 >= _NBUF))
        def _wait_store():
            pltpu.make_async_copy(
                bufs.at[nbo],
                out_hbm.at[pl.ds(bbase, _MB)],
                ssem.at[nbo],
            ).wait()

        @pl.when(m + 1 < _NSTEPS)
        def _prefetch():
            fire_gathers(m + 1, nbo)

        pltpu.make_async_copy(
            out_hbm.at[pl.ds(bbase, _MB), pl.ds(0, _TM)],
            bufs.at[bo, pl.ds(0, _MB), pl.ds(0, _TM)],
            gsem.at[bo],
        ).wait()
        pltpu.make_async_copy(
            out_hbm.at[pl.ds(bbase, 1), pl.ds(0, _L)],
            tbuf.at[bo],
            gsem.at[bo],
        ).wait()
        for k in range(_MB):
            for r in range(2):
                for h in range(2):
                    bufs[bo, k, _TM + r, pl.ds(h * _L, _L)] = tbuf[
                        bo, 2 * k + r, pl.ds(h * _L, _L)
                    ]
        pltpu.async_copy(
            bufs.at[bo],
            out_hbm.at[pl.ds(bbase + m * _MB, _MB)],
            ssem.at[bo],
        )
        return carry

    lax.fori_loop(0, _NSTEPS, outer, 0)

    for bo in range(_NBUF):
        pltpu.make_async_copy(
            bufs.at[bo],
            out_hbm.at[pl.ds(bbase, _MB)],
            ssem.at[bo],
        ).wait()


_gather = pl.kernel(
    _gather_body,
    out_type=jax.ShapeDtypeStruct((_B, _T, _D), jnp.float32),
    mesh=plsc.VectorSubcoreMesh(core_axis_name="c", subcore_axis_name="s"),
    scratch_types=[
        pltpu.VMEM((_NSTEPS * _SEC,), jnp.int32),
        pltpu.VMEM((_NBUF, _MB, _T, _D), jnp.float32),
        pltpu.VMEM((_NBUF, _L, _D), jnp.float32),
        pltpu.SemaphoreType.DMA((_NBUF,)),
        pltpu.SemaphoreType.DMA((_NBUF,)),
    ],
    compiler_params=pltpu.CompilerParams(use_tc_tiling_on_sc=False),
)


def kernel(idx, token_embedding_table):
    B, T = idx.shape
    idx = idx.astype(jnp.int32)
    main = idx[:, :_TM].reshape(_B // _MB, _MB * _TM)
    tails = idx[:, _TM:_T].reshape(_B // _MB, _MB * 2)
    gstream = jnp.concatenate([main, tails], axis=1).reshape(-1)
    tab16 = token_embedding_table.reshape(16, _GCOLS)
    c_lin = _prep(token_embedding_table, tab16)
    c = c_lin.reshape(_T * _VOCAB, _D)
    return _gather(c, gstream)
